# baseline (speedup 1.0000x reference)
import functools

import jax
import jax.numpy as jnp
from jax import lax
from jax.experimental import pallas as pl
from jax.experimental.pallas import tpu as pltpu
from jax.experimental.pallas import tpu_sc as plsc

B, S, D = 4, 2048, 1024
LANES = 16
VPR = D // LANES

S_TC = 1536
S_SC = S - S_TC
BS_TC = 1536

NC, NS = 2, 16
NW = NC * NS
S_PER_W = S_SC // NW
CH = 8
N_CH = S_PER_W // CH
NBUF = 3


def _sc_body(x_hbm, emb_hbm, out_hbm, emb_v, x_v, e_sem, x_sem, o_sem):
    wid = lax.axis_index("s") * NC + lax.axis_index("c")
    s0 = wid * S_PER_W

    def issue_in(ci, p):
        r = s0 + ci * CH
        ed = pltpu.async_copy(
            emb_hbm.at[pl.ds(S_TC + r, CH)], emb_v.at[p], e_sem.at[p]
        )
        xd = pltpu.async_copy(
            x_hbm.at[:, pl.ds(S_TC + r, CH), :], x_v.at[p], x_sem.at[p]
        )
        return ed, xd

    def issue_out(ci, p):
        r = s0 + ci * CH
        return pltpu.async_copy(
            x_v.at[p], out_hbm.at[:, pl.ds(r, CH), :], o_sem.at[p]
        )

    pend_in = {ci: issue_in(ci, ci % NBUF) for ci in range(min(2, N_CH))}
    pend_out = {}
    for ci in range(N_CH):
        p = ci % NBUF
        ed, xd = pend_in.pop(ci)
        ed.wait()
        xd.wait()

        @plsc.parallel_loop(0, CH * VPR, unroll=4)
        def _(j):
            r = lax.shift_right_logical(j, 6)
            c = lax.bitwise_and(j, VPR - 1)
            sl = pl.ds(c * LANES, LANES)
            ev = emb_v[p, r, sl]
            for b in range(B):
                x_v[p, b, r, sl] = x_v[p, b, r, sl] + ev

        pend_out[ci] = issue_out(ci, p)
        if ci + 2 < N_CH:
            if ci - 1 in pend_out:
                pend_out.pop(ci - 1).wait()
            pend_in[ci + 2] = issue_in(ci + 2, (ci + 2) % NBUF)
    for od in pend_out.values():
        od.wait()


@functools.cache
def _make_sc_add():
    return pl.kernel(
        _sc_body,
        out_type=jax.ShapeDtypeStruct((B, S_SC, D), jnp.float32),
        mesh=plsc.VectorSubcoreMesh(
            core_axis_name="c", subcore_axis_name="s", num_cores=NC, num_subcores=NS
        ),
        scratch_types=[
            pltpu.VMEM((NBUF, CH, D), jnp.float32),
            pltpu.VMEM((NBUF, B, CH, D), jnp.float32),
            pltpu.SemaphoreType.DMA((NBUF,)),
            pltpu.SemaphoreType.DMA((NBUF,)),
            pltpu.SemaphoreType.DMA((NBUF,)),
        ],
    )


def _tc_add_body(x_ref, emb_ref, o_ref):
    o_ref[...] = x_ref[...] + emb_ref[...][None]


def _tc_add(x, emb):
    grid = (S_TC // BS_TC, B)
    return pl.pallas_call(
        _tc_add_body,
        grid=grid,
        in_specs=[
            pl.BlockSpec((1, BS_TC, D), lambda i, j: (j, i, 0)),
            pl.BlockSpec((BS_TC, D), lambda i, j: (i, 0)),
        ],
        out_specs=pl.BlockSpec((1, BS_TC, D), lambda i, j: (j, i, 0)),
        out_shape=jax.ShapeDtypeStruct((B, S, D), x.dtype),
    )(x, emb)


def _stitch_body(tc_ref, sc_ref, o_ref):
    o_ref[...] = sc_ref[...]


def _stitch(tc_out, sc_out):
    return pl.pallas_call(
        _stitch_body,
        grid=(B,),
        in_specs=[
            pl.BlockSpec(memory_space=pl.ANY),
            pl.BlockSpec((1, S_SC, D), lambda j: (j, 0, 0)),
        ],
        out_specs=pl.BlockSpec((1, S_SC, D), lambda j: (j, S_TC // S_SC, 0)),
        out_shape=jax.ShapeDtypeStruct((B, S, D), tc_out.dtype),
        input_output_aliases={0: 0},
    )(tc_out, sc_out)


def kernel(x, emb):
    sc_out = _make_sc_add()(x, emb)
    tc_out = _tc_add(x, emb)
    return _stitch(tc_out, sc_out)


# speedup vs baseline: 1.0312x; 1.0312x over previous
import functools

import jax
import jax.numpy as jnp
from jax import lax
from jax.experimental import pallas as pl
from jax.experimental.pallas import tpu as pltpu
from jax.experimental.pallas import tpu_sc as plsc

B, S, D = 4, 2048, 1024
LANES = 16
VPR = D // LANES

S_TC = 1792
S_SC = S - S_TC
BS_TC = 1792

NC, NS = 2, 16
NW = NC * NS
S_PER_W = S_SC // NW
CH = 8
N_CH = S_PER_W // CH
NBUF = 3


def _sc_body(x_hbm, emb_hbm, out_hbm, emb_v, x_v, e_sem, x_sem, o_sem):
    wid = lax.axis_index("s") * NC + lax.axis_index("c")
    s0 = wid * S_PER_W

    def issue_in(ci, p):
        r = s0 + ci * CH
        ed = pltpu.async_copy(
            emb_hbm.at[pl.ds(S_TC + r, CH)], emb_v.at[p], e_sem.at[p]
        )
        xd = pltpu.async_copy(
            x_hbm.at[:, pl.ds(S_TC + r, CH), :], x_v.at[p], x_sem.at[p]
        )
        return ed, xd

    def issue_out(ci, p):
        r = s0 + ci * CH
        return pltpu.async_copy(
            x_v.at[p], out_hbm.at[:, pl.ds(r, CH), :], o_sem.at[p]
        )

    pend_in = {ci: issue_in(ci, ci % NBUF) for ci in range(min(2, N_CH))}
    pend_out = {}
    for ci in range(N_CH):
        p = ci % NBUF
        ed, xd = pend_in.pop(ci)
        ed.wait()
        xd.wait()

        @plsc.parallel_loop(0, CH * VPR, unroll=4)
        def _(j):
            r = lax.shift_right_logical(j, 6)
            c = lax.bitwise_and(j, VPR - 1)
            sl = pl.ds(c * LANES, LANES)
            ev = emb_v[p, r, sl]
            for b in range(B):
                x_v[p, b, r, sl] = x_v[p, b, r, sl] + ev

        pend_out[ci] = issue_out(ci, p)
        if ci + 2 < N_CH:
            if ci - 1 in pend_out:
                pend_out.pop(ci - 1).wait()
            pend_in[ci + 2] = issue_in(ci + 2, (ci + 2) % NBUF)
    for od in pend_out.values():
        od.wait()


@functools.cache
def _make_sc_add():
    return pl.kernel(
        _sc_body,
        out_type=jax.ShapeDtypeStruct((B, S_SC, D), jnp.float32),
        mesh=plsc.VectorSubcoreMesh(
            core_axis_name="c", subcore_axis_name="s", num_cores=NC, num_subcores=NS
        ),
        scratch_types=[
            pltpu.VMEM((NBUF, CH, D), jnp.float32),
            pltpu.VMEM((NBUF, B, CH, D), jnp.float32),
            pltpu.SemaphoreType.DMA((NBUF,)),
            pltpu.SemaphoreType.DMA((NBUF,)),
            pltpu.SemaphoreType.DMA((NBUF,)),
        ],
    )


def _tc_add_body(x_ref, emb_ref, o_ref):
    o_ref[...] = x_ref[...] + emb_ref[...][None]


def _tc_add(x, emb):
    grid = (S_TC // BS_TC, B)
    return pl.pallas_call(
        _tc_add_body,
        grid=grid,
        in_specs=[
            pl.BlockSpec((1, BS_TC, D), lambda i, j: (j, i, 0)),
            pl.BlockSpec((BS_TC, D), lambda i, j: (i, 0)),
        ],
        out_specs=pl.BlockSpec((1, BS_TC, D), lambda i, j: (j, i, 0)),
        out_shape=jax.ShapeDtypeStruct((B, S, D), x.dtype),
    )(x, emb)


def _stitch_body(tc_ref, sc_ref, o_ref):
    o_ref[...] = sc_ref[...]


def _stitch(tc_out, sc_out):
    return pl.pallas_call(
        _stitch_body,
        grid=(B,),
        in_specs=[
            pl.BlockSpec(memory_space=pl.ANY),
            pl.BlockSpec((1, S_SC, D), lambda j: (j, 0, 0)),
        ],
        out_specs=pl.BlockSpec((1, S_SC, D), lambda j: (j, S_TC // S_SC, 0)),
        out_shape=jax.ShapeDtypeStruct((B, S, D), tc_out.dtype),
        input_output_aliases={0: 0},
    )(tc_out, sc_out)


def kernel(x, emb):
    sc_out = _make_sc_add()(x, emb)
    tc_out = _tc_add(x, emb)
    return _stitch(tc_out, sc_out)
